# Initial kernel scaffold; baseline (speedup 1.0000x reference)
import jax
import jax.numpy as jnp
from jax.experimental import pallas as pl


def kernel(x, token_table, pos_table):
    raise NotImplementedError("write your pallas kernel here")


# speedup vs baseline: 3.4375x; 3.4375x over previous
import functools

import jax
import jax.numpy as jnp
from jax import lax
from jax.experimental import pallas as pl
from jax.experimental.pallas import tpu as pltpu
from jax.experimental.pallas import tpu_sc as plsc

VOCAB_SIZE = 100000
MAXLEN = 200
EMBED_DIM = 64
BATCH = 4096

NUM_WORKERS = 32
ROWS_PER_WORKER = (BATCH * MAXLEN) // NUM_WORKERS
CHUNK_ROWS = 2 * MAXLEN
CHUNKS_PER_WORKER = ROWS_PER_WORKER // CHUNK_ROWS
GATHER_SPLIT = 4
GATHER_ROWS = CHUNK_ROWS // GATHER_SPLIT
LANES = 16
D_SLICES = EMBED_DIM // LANES


def _body(x_ref, tab_ref, pos_ref, out_ref, idx_v, rows_v, pos_v, gsem):
    c = lax.axis_index("c")
    s = lax.axis_index("s")
    wid = s * 2 + c

    pltpu.sync_copy(pos_ref, pos_v)

    def chunk_body(g, carry):
        cid = wid * CHUNKS_PER_WORKER + g
        pltpu.sync_copy(x_ref.at[cid], idx_v)
        cps = [
            pltpu.async_copy(
                tab_ref.at[idx_v.at[i]],
                rows_v.at[pl.ds(i * GATHER_ROWS, GATHER_ROWS)],
                gsem,
            )
            for i in range(GATHER_SPLIT)
        ]
        for cp in cps:
            cp.wait()

        def add_body(j, c2):
            for d in range(D_SLICES):
                p = pos_v[j, pl.ds(d * LANES, LANES)]
                for b in range(CHUNK_ROWS // MAXLEN):
                    plsc.addupdate(
                        rows_v.at[b * MAXLEN + j, pl.ds(d * LANES, LANES)], p
                    )
            return c2

        lax.fori_loop(0, MAXLEN, add_body, 0)

        base = wid * ROWS_PER_WORKER + g * CHUNK_ROWS
        pltpu.sync_copy(rows_v, out_ref.at[pl.ds(base, CHUNK_ROWS)])
        return carry

    lax.fori_loop(0, CHUNKS_PER_WORKER, chunk_body, 0)


@jax.jit
def kernel(x, token_table, pos_table):
    x_r = x.reshape(-1).astype(jnp.int32).reshape(
        NUM_WORKERS * CHUNKS_PER_WORKER, GATHER_SPLIT, GATHER_ROWS
    )
    mesh = plsc.VectorSubcoreMesh(core_axis_name="c", subcore_axis_name="s")
    run = functools.partial(
        pl.kernel,
        mesh=mesh,
        out_type=jax.ShapeDtypeStruct((BATCH * MAXLEN, EMBED_DIM), jnp.float32),
        scratch_types=[
            pltpu.VMEM((GATHER_SPLIT, GATHER_ROWS), jnp.int32),
            pltpu.VMEM((CHUNK_ROWS, EMBED_DIM), jnp.float32),
            pltpu.VMEM((MAXLEN, EMBED_DIM), jnp.float32),
            pltpu.SemaphoreType.DMA,
        ],
        compiler_params=pltpu.CompilerParams(use_tc_tiling_on_sc=False),
    )(_body)
    out = run(x_r, token_table, pos_table)
    return out.reshape(BATCH, MAXLEN, EMBED_DIM)


# speedup vs baseline: 4.1467x; 1.2063x over previous
import functools

import jax
import jax.numpy as jnp
from jax import lax
from jax.experimental import pallas as pl
from jax.experimental.pallas import tpu as pltpu
from jax.experimental.pallas import tpu_sc as plsc

VOCAB_SIZE = 100000
MAXLEN = 200
EMBED_DIM = 64
BATCH = 4096

NUM_WORKERS = 32
ROWS_PER_WORKER = (BATCH * MAXLEN) // NUM_WORKERS
CHUNK_ROWS = 2 * MAXLEN
CHUNKS_PER_WORKER = ROWS_PER_WORKER // CHUNK_ROWS
GATHER_SPLIT = 4
GATHER_ROWS = CHUNK_ROWS // GATHER_SPLIT
LANES = 16
D_SLICES = EMBED_DIM // LANES


def _body(x_ref, tab_ref, pos_ref, out_ref, idx_v, rows_v, pos_v,
          gsem0, gsem1, osem0, osem1):
    c = lax.axis_index("c")
    s = lax.axis_index("s")
    wid = s * 2 + c
    gsem = (gsem0, gsem1)
    osem = (osem0, osem1)

    pltpu.sync_copy(pos_ref, pos_v)
    pltpu.sync_copy(
        x_ref.at[pl.ds(wid * CHUNKS_PER_WORKER, CHUNKS_PER_WORKER)], idx_v
    )

    def fire_gathers(g, buf, sem):
        for i in range(GATHER_SPLIT):
            pltpu.async_copy(
                tab_ref.at[idx_v.at[g, i]],
                rows_v.at[buf, pl.ds(i * GATHER_ROWS, GATHER_ROWS)],
                sem,
            )

    fire_gathers(0, 0, gsem[0])

    def outer(i, carry):
        for b in range(2):
            g = 2 * i + b
            nb = 1 - b

            @pl.when(g < CHUNKS_PER_WORKER - 1)
            def _prefetch():
                @pl.when(g >= 1)
                def _drain_out():
                    pltpu.make_async_copy(
                        rows_v.at[nb], out_ref.at[pl.ds(0, CHUNK_ROWS)],
                        osem[nb],
                    ).wait()
                fire_gathers(g + 1, nb, gsem[nb])

            pltpu.make_async_copy(
                tab_ref.at[pl.ds(0, CHUNK_ROWS)], rows_v.at[b], gsem[b]
            ).wait()

            def add_body(j, c2):
                for d in range(D_SLICES):
                    p = pos_v[j, pl.ds(d * LANES, LANES)]
                    for r in range(CHUNK_ROWS // MAXLEN):
                        plsc.addupdate(
                            rows_v.at[b, r * MAXLEN + j,
                                      pl.ds(d * LANES, LANES)],
                            p,
                        )
                return c2

            lax.fori_loop(0, MAXLEN, add_body, 0, unroll=8)

            base = wid * ROWS_PER_WORKER + g * CHUNK_ROWS
            pltpu.async_copy(
                rows_v.at[b], out_ref.at[pl.ds(base, CHUNK_ROWS)], osem[b]
            )
        return carry

    lax.fori_loop(0, CHUNKS_PER_WORKER // 2, outer, 0)

    for b in range(2):
        pltpu.make_async_copy(
            rows_v.at[b], out_ref.at[pl.ds(0, CHUNK_ROWS)], osem[b]
        ).wait()


@jax.jit
def kernel(x, token_table, pos_table):
    x_r = x.reshape(-1).astype(jnp.int32).reshape(
        NUM_WORKERS * CHUNKS_PER_WORKER, GATHER_SPLIT, GATHER_ROWS
    )
    mesh = plsc.VectorSubcoreMesh(core_axis_name="c", subcore_axis_name="s")
    run = functools.partial(
        pl.kernel,
        mesh=mesh,
        out_type=jax.ShapeDtypeStruct((BATCH * MAXLEN, EMBED_DIM), jnp.float32),
        scratch_types=[
            pltpu.VMEM((CHUNKS_PER_WORKER, GATHER_SPLIT, GATHER_ROWS),
                       jnp.int32),
            pltpu.VMEM((2, CHUNK_ROWS, EMBED_DIM), jnp.float32),
            pltpu.VMEM((MAXLEN, EMBED_DIM), jnp.float32),
            pltpu.SemaphoreType.DMA,
            pltpu.SemaphoreType.DMA,
            pltpu.SemaphoreType.DMA,
            pltpu.SemaphoreType.DMA,
        ],
        compiler_params=pltpu.CompilerParams(use_tc_tiling_on_sc=False),
    )(_body)
    out = run(x_r, token_table, pos_table)
    return out.reshape(BATCH, MAXLEN, EMBED_DIM)
